# baseline (speedup 1.0000x reference)
import dataclasses

import numpy as np

import jax
import jax.numpy as jnp
from jax import lax
from jax.experimental import pallas as pl
from jax.experimental.pallas import tpu as pltpu
from jax.experimental.pallas import tpu_sc as plsc

NC = 2
NS = 16
NW = NC * NS
L = 16
CHUNK = 128
BM = 8
OUT_RB = 624


def _unpack_perm(d):
    perm = np.zeros(d, dtype=np.int32)
    for g in range(d // 32):
        for i in range(16):
            perm[32 * g + i] = 32 * g + 2 * i
            perm[32 * g + 16 + i] = 32 * g + 2 * i + 1
    return perm


def _pack_table(x):
    n, d = x.shape
    xb = x.astype(jnp.bfloat16)
    return jax.lax.bitcast_convert_type(
        xb.reshape(n, d // 2, 2), jnp.int32)


def _sc_aggregate(n, d, r_total, with_deg):
    rpt_pair = r_total // NS
    rpt0 = rpt_pair // 2
    rpt1 = rpt_pair - rpt0
    assert rpt0 % (2 * BM) == 0 and rpt1 % (2 * BM) == 0
    assert n == 15 * OUT_RB + OUT_RB + 16
    deg_chunk = 1024
    n_deg = 10 * deg_chunk

    mesh = plsc.VectorSubcoreMesh(core_axis_name="c", subcore_axis_name="s")

    out_type = [jax.ShapeDtypeStruct((NC, n, d), jnp.float32)]
    if with_deg:
        out_type = out_type + [
            jax.ShapeDtypeStruct((NC * n_deg,), jnp.float32)]

    meta_block = [
        pltpu.VMEM((BM, CHUNK), jnp.int32),
        pltpu.VMEM((BM, CHUNK), jnp.int32),
        pltpu.VMEM((BM, CHUNK), jnp.float32),
    ]
    scratch_types = meta_block + meta_block + [
        pltpu.VMEM((CHUNK, d // 2), jnp.int32),
        pltpu.VMEM((CHUNK, d // 2), jnp.int32),
        pltpu.VMEM((CHUNK, d), jnp.float32),
        pltpu.VMEM((CHUNK,), jnp.float32),
        pltpu.VMEM((1024,), jnp.float32),
        pltpu.VMEM_SHARED((n, d), jnp.float32),
        pltpu.VMEM_SHARED((n_deg,), jnp.float32),
        pltpu.SemaphoreType.DMA,
        pltpu.SemaphoreType.DMA,
        pltpu.SemaphoreType.DMA,
        pltpu.SemaphoreType.DMA,
        pltpu.SemaphoreType.DMA,
        pltpu.SemaphoreType.DMA,
    ]

    def body(*refs):
        if with_deg:
            (x_hbm, src_hbm, dst_hbm, w_hbm, agg_out, deg_out,
             srcA, dstA, wA, srcB, dstB, wB,
             rows_p0, rows_p1, rows_f, mask_v, z1_v, agg_sh, deg_sh,
             msemA, msemB, gsem0, gsem1, ssem, dsem) = refs
        else:
            (x_hbm, src_hbm, dst_hbm, w_hbm, agg_out,
             srcA, dstA, wA, srcB, dstB, wB,
             rows_p0, rows_p1, rows_f, mask_v, z1_v, agg_sh, deg_sh,
             msemA, msemB, gsem0, gsem1, ssem, dsem) = refs

        c = lax.axis_index("c")
        s = lax.axis_index("s")
        ebase = jnp.where(c == 0, s * rpt0, NS * rpt0 + s * rpt1)
        nblocks = jnp.where(c == 0, rpt0 // BM, rpt1 // BM)

        def meta_descs(b, bufs, sem):
            sl = pl.ds(ebase + b * BM, BM)
            return [
                pltpu.make_async_copy(src_hbm.at[sl], bufs[0], sem),
                pltpu.make_async_copy(dst_hbm.at[sl], bufs[1], sem),
                pltpu.make_async_copy(w_hbm.at[sl], bufs[2], sem),
            ]

        bufsA = (srcA, dstA, wA)
        bufsB = (srcB, dstB, wB)

        for desc in meta_descs(0, bufsA, msemA):
            desc.start()

        zero16 = jnp.zeros((L,), jnp.float32)

        @pl.loop(0, CHUNK)
        def _(i):
            for j in range(d // L):
                rows_f[i, pl.ds(j * L, L)] = zero16

        for k in range(4):
            pltpu.sync_copy(rows_f,
                            agg_sh.at[pl.ds(s * OUT_RB + k * CHUNK, CHUNK)])
        pltpu.sync_copy(rows_f.at[pl.ds(0, OUT_RB - 4 * CHUNK)],
                        agg_sh.at[pl.ds(s * OUT_RB + 4 * CHUNK,
                                        OUT_RB - 4 * CHUNK)])

        @pl.when(s == NS - 1)
        def _():
            pltpu.sync_copy(rows_f.at[pl.ds(0, 16)],
                            agg_sh.at[pl.ds(16 * OUT_RB, 16)])

        if with_deg:
            @pl.loop(0, 1024 // L)
            def _(i):
                z1_v[pl.ds(i * L, L)] = zero16

            @pl.when(s < n_deg // deg_chunk)
            def _():
                pltpu.sync_copy(z1_v,
                                deg_sh.at[pl.ds(s * deg_chunk, deg_chunk)])

        plsc.subcore_barrier()

        def g_desc(src_ref, buf, sem):
            return pltpu.make_async_copy(x_hbm.at[src_ref], buf, sem)

        def s_desc(dst_ref):
            return pltpu.make_async_copy(rows_f, agg_sh.at[dst_ref], ssem)

        def d_desc(dst_ref):
            return pltpu.make_async_copy(mask_v, deg_sh.at[dst_ref], dsem)

        def scale(pbuf, w_b, cidx):
            @pl.loop(0, CHUNK // L)
            def _(i16):
                w16 = w_b[cidx, pl.ds(i16 * L, L)]
                if with_deg:
                    mask_v[pl.ds(i16 * L, L)] = jnp.where(
                        w16 != 0.0, 1.0, 0.0)
                for ii in range(L):
                    wv = w16[ii]
                    row = i16 * L + ii
                    for g in range(d // (2 * L)):
                        pk = pbuf[row, pl.ds(g * L, L)]
                        ab = plsc.bitcast(pk, jnp.bfloat16)
                        ea, eb = plsc.unpack(
                            ab, format=plsc.PackFormat.INTERLEAVED)
                        rows_f[row, pl.ds(g * 2 * L, L)] = ea * wv
                        rows_f[row, pl.ds(g * 2 * L + L, L)] = eb * wv

        def process_block(b, cur, cur_sem, nxt, nxt_sem):
            src_b, dst_b, w_b = cur
            for desc in meta_descs(b, cur, cur_sem):
                desc.wait()

            @pl.when(b > 0)
            def _():
                s_desc(dst_b.at[0]).wait()
                if with_deg:
                    d_desc(dst_b.at[0]).wait()

            @pl.when(b + 1 < nblocks)
            def _():
                for desc in meta_descs(b + 1, nxt, nxt_sem):
                    desc.start()

            g_desc(src_b.at[0], rows_p0, gsem0).start()
            g_desc(src_b.at[1], rows_p1, gsem1).start()

            @pl.loop(0, BM, step=2)
            def _(k):
                g_desc(src_b.at[k], rows_p0, gsem0).wait()

                @pl.when(k > 0)
                def _():
                    s_desc(dst_b.at[0]).wait()
                    if with_deg:
                        d_desc(dst_b.at[0]).wait()

                scale(rows_p0, w_b, k)
                s_desc(dst_b.at[k]).start(add=True)
                if with_deg:
                    d_desc(dst_b.at[k]).start(add=True)

                @pl.when(k + 2 < BM)
                def _():
                    g_desc(src_b.at[k + 2], rows_p0, gsem0).start()

                g_desc(src_b.at[k + 1], rows_p1, gsem1).wait()
                s_desc(dst_b.at[0]).wait()
                if with_deg:
                    d_desc(dst_b.at[0]).wait()
                scale(rows_p1, w_b, k + 1)
                s_desc(dst_b.at[k + 1]).start(add=True)
                if with_deg:
                    d_desc(dst_b.at[k + 1]).start(add=True)

                @pl.when(k + 3 < BM)
                def _():
                    g_desc(src_b.at[k + 3], rows_p1, gsem1).start()

        @pl.loop(0, nblocks, step=2)
        def _(b):
            process_block(b, bufsA, msemA, bufsB, msemB)
            process_block(b + 1, bufsB, msemB, bufsA, msemA)

        s_desc(dstA.at[0]).wait()
        if with_deg:
            d_desc(dstA.at[0]).wait()

        plsc.subcore_barrier()

        pltpu.sync_copy(agg_sh.at[pl.ds(s * OUT_RB, OUT_RB)],
                        agg_out.at[c, pl.ds(s * OUT_RB, OUT_RB)])

        @pl.when(s == NS - 1)
        def _():
            pltpu.sync_copy(agg_sh.at[pl.ds(16 * OUT_RB, 16)],
                            agg_out.at[c, pl.ds(16 * OUT_RB, 16)])

        if with_deg:
            @pl.when(s < n_deg // deg_chunk)
            def _():
                pltpu.sync_copy(
                    deg_sh.at[pl.ds(s * deg_chunk, deg_chunk)],
                    deg_out.at[pl.ds(c * n_deg + s * deg_chunk, deg_chunk)])

    cp = pltpu.CompilerParams(needs_layout_passes=False,
                              use_tc_tiling_on_sc=False)
    return pl.kernel(body, out_type=out_type, mesh=mesh,
                     scratch_types=scratch_types, compiler_params=cp)


def _tc_layer(x, agg_p, deg0, deg1, w_self, w_neigh_perm, b2d, relu):
    n, d = x.shape
    rb = 1000

    def body(x_ref, a_ref, g0_ref, g1_ref, ws_ref, wn_ref, b_ref, o_ref):
        deg = jnp.maximum(g0_ref[...] + g1_ref[...], 1.0)
        hn = (a_ref[0] + a_ref[1]) / deg
        acc = (
            jnp.dot(x_ref[...], ws_ref[...],
                    preferred_element_type=jnp.float32)
            + jnp.dot(hn, wn_ref[...],
                      preferred_element_type=jnp.float32)
            + b_ref[...])
        o_ref[...] = jnp.maximum(acc, 0.0) if relu else acc

    return pl.pallas_call(
        body,
        grid=(n // rb,),
        in_specs=[
            pl.BlockSpec((rb, d), lambda i: (i, 0)),
            pl.BlockSpec((2, rb, d), lambda i: (0, i, 0)),
            pl.BlockSpec((rb, 1), lambda i: (i, 0)),
            pl.BlockSpec((rb, 1), lambda i: (i, 0)),
            pl.BlockSpec((d, d), lambda i: (0, 0)),
            pl.BlockSpec((d, d), lambda i: (0, 0)),
            pl.BlockSpec((1, d), lambda i: (0, 0)),
        ],
        out_specs=pl.BlockSpec((rb, d), lambda i: (i, 0)),
        out_shape=jax.ShapeDtypeStruct((n, d), jnp.float32),
    )(x, agg_p, deg0, deg1, w_self, w_neigh_perm, b2d)


def kernel(inputs, edge_index, edge_weight, W_self1, W_neigh1, b1,
           W_self2, W_neigh2, b2):
    x = inputs
    n, d = x.shape
    e = edge_index.shape[1]

    gran = NS * CHUNK * 4 * BM
    epad = ((e + gran - 1) // gran) * gran
    p = epad - e
    pad_idx = jnp.arange(p, dtype=jnp.int32)
    src = jnp.concatenate([edge_index[0], pad_idx % n])
    dst = jnp.concatenate([edge_index[1], pad_idx % n])
    w = jnp.pad(edge_weight, (0, p))
    r_total = epad // CHUNK
    src2d = src.reshape(r_total, CHUNK)
    dst2d = dst.reshape(r_total, CHUNK)
    w2d = w.reshape(r_total, CHUNK)

    perm = _unpack_perm(d)
    wn1 = W_neigh1[perm, :]
    wn2 = W_neigh2[perm, :]

    sc1 = _sc_aggregate(n, d, r_total, with_deg=True)
    sc2 = _sc_aggregate(n, d, r_total, with_deg=False)

    agg_p, deg_p = sc1(_pack_table(x), src2d, dst2d, w2d)
    deg_flat = deg_p.reshape(NC, -1)
    deg0 = deg_flat[0, :n].reshape(n, 1)
    deg1 = deg_flat[1, :n].reshape(n, 1)
    b1r = b1.reshape(1, d)
    b2r = b2.reshape(1, d)

    h = _tc_layer(x, agg_p, deg0, deg1, W_self1, wn1, b1r, relu=True)
    (agg2_p,) = sc2(_pack_table(h), src2d, dst2d, w2d)
    out = _tc_layer(h, agg2_p, deg0, deg1, W_self2, wn2, b2r, relu=False)
    return out


# speedup vs baseline: 2.0009x; 2.0009x over previous
import jax
import jax.numpy as jnp
from jax import lax
from jax.experimental import pallas as pl
from jax.experimental.pallas import tpu as pltpu
from jax.experimental.pallas import tpu_sc as plsc

NC = 2
NS = 16
NW = NC * NS
L = 16
CHUNK = 128
BM = 8
OUT_RB = 624


def _sc_aggregate(n, d, r_total, with_deg):
    rpt_pair = r_total // NS
    rpt0 = rpt_pair // 2
    rpt1 = rpt_pair - rpt0
    assert rpt0 % (2 * BM) == 0 and rpt1 % (2 * BM) == 0
    assert n == 15 * OUT_RB + OUT_RB + 16
    deg_chunk = 1024
    n_deg = 10 * deg_chunk

    mesh = plsc.VectorSubcoreMesh(core_axis_name="c", subcore_axis_name="s")

    out_type = [jax.ShapeDtypeStruct((NC, n, d), jnp.float32)]
    if with_deg:
        out_type = out_type + [
            jax.ShapeDtypeStruct((NC * n_deg,), jnp.float32)]

    meta_block = [
        pltpu.VMEM((BM, CHUNK), jnp.int32),
        pltpu.VMEM((BM, CHUNK), jnp.int32),
        pltpu.VMEM((BM, CHUNK), jnp.float32),
        pltpu.VMEM((BM, CHUNK), jnp.float32),
    ]
    scratch_types = meta_block + meta_block + [
        pltpu.VMEM((CHUNK, d), jnp.float32),
        pltpu.VMEM((CHUNK, d), jnp.float32),
        pltpu.VMEM((1024,), jnp.float32),
        pltpu.VMEM_SHARED((n + CHUNK, d), jnp.float32),
        pltpu.VMEM_SHARED((n_deg,), jnp.float32),
        pltpu.SemaphoreType.DMA,
        pltpu.SemaphoreType.DMA,
        pltpu.SemaphoreType.DMA,
        pltpu.SemaphoreType.DMA,
        pltpu.SemaphoreType.DMA,
        pltpu.SemaphoreType.DMA,
        pltpu.SemaphoreType.DMA,
    ]

    def body(*refs):
        if with_deg:
            (x_hbm, src_hbm, dst_hbm, w_hbm, mask_hbm, agg_out, deg_out,
             srcA, dstA, wA, maskA, srcB, dstB, wB, maskB,
             rows0, rows1, z1_v, agg_sh, deg_sh,
             msemA, msemB, gsem0, gsem1, ssem0, ssem1, dsem) = refs
        else:
            (x_hbm, src_hbm, dst_hbm, w_hbm, agg_out,
             srcA, dstA, wA, maskA, srcB, dstB, wB, maskB,
             rows0, rows1, z1_v, agg_sh, deg_sh,
             msemA, msemB, gsem0, gsem1, ssem0, ssem1, dsem) = refs

        c = lax.axis_index("c")
        s = lax.axis_index("s")
        ebase = jnp.where(c == 0, s * rpt0, NS * rpt0 + s * rpt1)
        nblocks = jnp.where(c == 0, rpt0 // BM, rpt1 // BM)

        def meta_descs(b, bufs, sem):
            sl = pl.ds(ebase + b * BM, BM)
            descs = [
                pltpu.make_async_copy(src_hbm.at[sl], bufs[0], sem),
                pltpu.make_async_copy(dst_hbm.at[sl], bufs[1], sem),
                pltpu.make_async_copy(w_hbm.at[sl], bufs[2], sem),
            ]
            if with_deg:
                descs.append(
                    pltpu.make_async_copy(mask_hbm.at[sl], bufs[3], sem))
            return descs

        bufsA = (srcA, dstA, wA, maskA)
        bufsB = (srcB, dstB, wB, maskB)

        for desc in meta_descs(0, bufsA, msemA):
            desc.start()

        zero16 = jnp.zeros((L,), jnp.float32)

        @pl.loop(0, CHUNK)
        def _(i):
            for j in range(d // L):
                rows0[i, pl.ds(j * L, L)] = zero16

        for k in range(4):
            pltpu.sync_copy(rows0,
                            agg_sh.at[pl.ds(s * OUT_RB + k * CHUNK, CHUNK)])
        pltpu.sync_copy(rows0.at[pl.ds(0, OUT_RB - 4 * CHUNK)],
                        agg_sh.at[pl.ds(s * OUT_RB + 4 * CHUNK,
                                        OUT_RB - 4 * CHUNK)])

        @pl.when(s == NS - 1)
        def _():
            pltpu.sync_copy(rows0.at[pl.ds(0, 16)],
                            agg_sh.at[pl.ds(16 * OUT_RB, 16)])

        if with_deg:
            @pl.loop(0, 1024 // L)
            def _(i):
                z1_v[pl.ds(i * L, L)] = zero16

            @pl.when(s < n_deg // deg_chunk)
            def _():
                pltpu.sync_copy(z1_v,
                                deg_sh.at[pl.ds(s * deg_chunk, deg_chunk)])

        plsc.subcore_barrier()

        def g_desc(src_ref, buf, sem):
            return pltpu.make_async_copy(x_hbm.at[src_ref], buf, sem)

        def s_desc(buf, dst_ref, sem):
            return pltpu.make_async_copy(buf, agg_sh.at[dst_ref], sem)

        def d_desc(mask_ref, dst_ref):
            return pltpu.make_async_copy(mask_ref, deg_sh.at[dst_ref], dsem)

        def scale(buf, w_ref, cidx):
            @pl.loop(0, CHUNK // L)
            def _(i16):
                w16 = w_ref[cidx, pl.ds(i16 * L, L)]
                for ii in range(L):
                    wv = w16[ii]
                    row = i16 * L + ii
                    for jj in range(d // L):
                        sl = pl.ds(jj * L, L)
                        buf[row, sl] = buf[row, sl] * wv

        def process_block(b, cur, cur_sem, nxt, nxt_sem):
            src_b, dst_b, w_b, mask_b = cur
            for desc in meta_descs(b, cur, cur_sem):
                desc.wait()

            @pl.when(b > 0)
            def _():
                s_desc(rows0, dst_b.at[0], ssem0).wait()
                s_desc(rows1, dst_b.at[0], ssem1).wait()
                if with_deg:
                    for _ in range(BM):
                        d_desc(mask_b.at[0], dst_b.at[0]).wait()

            @pl.when(b + 1 < nblocks)
            def _():
                for desc in meta_descs(b + 1, nxt, nxt_sem):
                    desc.start()

            g_desc(src_b.at[0], rows0, gsem0).start()
            g_desc(src_b.at[1], rows1, gsem1).start()

            @pl.loop(0, BM, step=2)
            def _(k):
                g_desc(src_b.at[k], rows0, gsem0).wait()
                scale(rows0, w_b, k)
                s_desc(rows0, dst_b.at[k], ssem0).start(add=True)
                if with_deg:
                    d_desc(mask_b.at[k], dst_b.at[k]).start(add=True)

                g_desc(src_b.at[k + 1], rows1, gsem1).wait()
                scale(rows1, w_b, k + 1)
                s_desc(rows1, dst_b.at[k + 1], ssem1).start(add=True)
                if with_deg:
                    d_desc(mask_b.at[k + 1], dst_b.at[k + 1]).start(add=True)

                @pl.when(k + 2 < BM)
                def _():
                    s_desc(rows0, dst_b.at[k], ssem0).wait()
                    g_desc(src_b.at[k + 2], rows0, gsem0).start()
                    s_desc(rows1, dst_b.at[k + 1], ssem1).wait()
                    g_desc(src_b.at[k + 3], rows1, gsem1).start()

        @pl.loop(0, nblocks, step=2)
        def _(b):
            process_block(b, bufsA, msemA, bufsB, msemB)
            process_block(b + 1, bufsB, msemB, bufsA, msemA)

        s_desc(rows0, dstA.at[0], ssem0).wait()
        s_desc(rows1, dstA.at[0], ssem1).wait()
        if with_deg:
            for _ in range(BM):
                d_desc(maskA.at[0], dstA.at[0]).wait()

        plsc.subcore_barrier()

        pltpu.sync_copy(agg_sh.at[pl.ds(s * OUT_RB, OUT_RB)],
                        agg_out.at[c, pl.ds(s * OUT_RB, OUT_RB)])

        @pl.when(s == NS - 1)
        def _():
            pltpu.sync_copy(agg_sh.at[pl.ds(16 * OUT_RB, 16)],
                            agg_out.at[c, pl.ds(16 * OUT_RB, 16)])

        if with_deg:
            @pl.when(s < n_deg // deg_chunk)
            def _():
                pltpu.sync_copy(
                    deg_sh.at[pl.ds(s * deg_chunk, deg_chunk)],
                    deg_out.at[pl.ds(c * n_deg + s * deg_chunk, deg_chunk)])

    return pl.kernel(body, out_type=out_type, mesh=mesh,
                     scratch_types=scratch_types)


def _tc_layer(x, agg_p, deg0, deg1, w_self, w_neigh, b2d, relu):
    n, d = x.shape
    rb = 1000

    def body(x_ref, a_ref, g0_ref, g1_ref, ws_ref, wn_ref, b_ref, o_ref):
        deg = jnp.maximum(g0_ref[...] + g1_ref[...], 1.0)
        hn = (a_ref[0] + a_ref[1]) / deg
        acc = (
            jnp.dot(x_ref[...], ws_ref[...],
                    preferred_element_type=jnp.float32)
            + jnp.dot(hn, wn_ref[...],
                      preferred_element_type=jnp.float32)
            + b_ref[...])
        o_ref[...] = jnp.maximum(acc, 0.0) if relu else acc

    return pl.pallas_call(
        body,
        grid=(n // rb,),
        in_specs=[
            pl.BlockSpec((rb, d), lambda i: (i, 0)),
            pl.BlockSpec((2, rb, d), lambda i: (0, i, 0)),
            pl.BlockSpec((rb, 1), lambda i: (i, 0)),
            pl.BlockSpec((rb, 1), lambda i: (i, 0)),
            pl.BlockSpec((d, d), lambda i: (0, 0)),
            pl.BlockSpec((d, d), lambda i: (0, 0)),
            pl.BlockSpec((1, d), lambda i: (0, 0)),
        ],
        out_specs=pl.BlockSpec((rb, d), lambda i: (i, 0)),
        out_shape=jax.ShapeDtypeStruct((n, d), jnp.float32),
    )(x, agg_p, deg0, deg1, w_self, w_neigh, b2d)


def kernel(inputs, edge_index, edge_weight, W_self1, W_neigh1, b1,
           W_self2, W_neigh2, b2):
    x = inputs
    n, d = x.shape
    e = edge_index.shape[1]

    gran = NS * CHUNK * 4 * BM
    epad = ((e + gran - 1) // gran) * gran
    p = epad - e
    pad_idx = jnp.arange(p, dtype=jnp.int32)
    src = jnp.concatenate([edge_index[0], pad_idx % n])
    dst = jnp.concatenate([edge_index[1], n + (pad_idx % CHUNK)])
    w = jnp.pad(edge_weight, (0, p))
    mask = jnp.pad(jnp.ones((e,), jnp.float32), (0, p))
    r_total = epad // CHUNK
    src2d = src.reshape(r_total, CHUNK)
    dst2d = dst.reshape(r_total, CHUNK)
    w2d = w.reshape(r_total, CHUNK)
    mask2d = mask.reshape(r_total, CHUNK)

    sc1 = _sc_aggregate(n, d, r_total, with_deg=True)
    sc2 = _sc_aggregate(n, d, r_total, with_deg=False)

    agg_p, deg_p = sc1(x, src2d, dst2d, w2d, mask2d)
    deg_flat = deg_p.reshape(NC, -1)
    deg0 = deg_flat[0, :n].reshape(n, 1)
    deg1 = deg_flat[1, :n].reshape(n, 1)
    b1r = b1.reshape(1, d)
    b2r = b2.reshape(1, d)

    h = _tc_layer(x, agg_p, deg0, deg1, W_self1, W_neigh1, b1r, relu=True)
    (agg2_p,) = sc2(h, src2d, dst2d, w2d)
    out = _tc_layer(h, agg2_p, deg0, deg1, W_self2, W_neigh2, b2r,
                    relu=False)
    return out
